# baseline (speedup 1.0000x reference)
import functools

import jax
import jax.numpy as jnp
from jax import lax
from jax.experimental import pallas as pl
from jax.experimental.pallas import tpu as pltpu
from jax.experimental.pallas import tpu_sc as plsc

_LANES = 16


def _make_gather(batch: int, fields: int, dim: int, num_workers: int):
  per_w = batch // num_workers
  pitch = per_w + 1
  mesh = plsc.VectorSubcoreMesh(core_axis_name="c", subcore_axis_name="s")
  nc = mesh.num_cores

  @functools.partial(
      pl.kernel,
      out_type=jax.ShapeDtypeStruct(
          (fields, dim // 8, batch // 128, 8, 128), jnp.float32),
      mesh=mesh,
      scratch_types=[
          pltpu.VMEM((2, per_w), jnp.int32),
          pltpu.VMEM((2, per_w, dim), jnp.float32),
          pltpu.VMEM((2, dim, pitch), jnp.float32),
          pltpu.SemaphoreType.DMA,
          pltpu.SemaphoreType.DMA,
          pltpu.SemaphoreType.DMA,
          pltpu.SemaphoreType.DMA,
          pltpu.SemaphoreType.DMA,
          pltpu.SemaphoreType.DMA,
      ],
      compiler_params=pltpu.CompilerParams(
          use_tc_tiling_on_sc=False, needs_layout_passes=False,
          disable_bounds_checks=True),
  )
  def gather_kernel(table_hbm, idxt_hbm, out_hbm, idx_v, rows_v, cols_v,
                    is0, is1, gs0, gs1, ss0, ss1):
    wid = lax.axis_index("s") * nc + lax.axis_index("c")
    b0 = wid * per_w
    lane = lax.iota(jnp.int32, _LANES)
    isem = (is0, is1)
    gsem = (gs0, gs1)
    ssem = (ss0, ss1)

    def fire_idx(f, p):
      pltpu.async_copy(idxt_hbm.at[f, pl.ds(b0, per_w)], idx_v.at[p], isem[p])

    def wait_idx(p):
      pltpu.make_async_copy(idxt_hbm.at[0, pl.ds(b0, per_w)], idx_v.at[p],
                            isem[p]).wait()

    def fire_gather(p):
      pltpu.async_copy(table_hbm.at[idx_v.at[p]], rows_v.at[p], gsem[p])

    def wait_gather(p):
      pltpu.make_async_copy(table_hbm.at[idx_v.at[p]], rows_v.at[p],
                            gsem[p]).wait()

    def fire_store(f, p):
      for dt in range(dim // 8):
        for btr in range(per_w // 128):
          pltpu.async_copy(
              cols_v.at[p, pl.ds(dt * 8, 8), pl.ds(btr * 128, 128)],
              out_hbm.at[f, dt, wid * (per_w // 128) + btr, :, :], ssem[p])

    def wait_store(p):
      for dt in range(dim // 8):
        for btr in range(per_w // 128):
          pltpu.make_async_copy(
              cols_v.at[p, pl.ds(dt * 8, 8), pl.ds(btr * 128, 128)],
              out_hbm.at[0, dt, wid * (per_w // 128) + btr, :, :],
              ssem[p]).wait()

    def transpose(p):
      @plsc.parallel_loop(0, per_w, step=1, unroll=16)
      def _t_loop(r):
        rsplat = jnp.full((_LANES,), 0, jnp.int32) + r
        for h in range(dim // _LANES):
          d_ids = lane + h * _LANES
          v = rows_v[p, r, pl.ds(h * _LANES, _LANES)]
          plsc.store_scatter(cols_v.at[p], [d_ids, rsplat], v)

    fire_idx(0, 0)
    fire_idx(1, 1)
    wait_idx(0)
    fire_gather(0)

    @pl.loop(0, fields // 2)
    def _o_loop(o):
      for q in (0, 1):
        f = 2 * o + q
        p = q
        @pl.when(f < fields - 1)
        def _():
          wait_idx(1 - p)
          fire_gather(1 - p)

        wait_gather(p)

        @pl.when(f >= 2)
        def _():
          wait_store(p)

        transpose(p)
        fire_store(f, p)

        @pl.when(f < fields - 2)
        def _():
          fire_idx(f + 2, p)

    wait_store(0)
    wait_store(1)

  return gather_kernel


def kernel(x, emb_weight):
  b, f = x.shape
  dim = emb_weight.shape[1]
  xt = jnp.transpose(x)
  out5 = _make_gather(b, f, dim, 32)(emb_weight, xt)
  o7 = jnp.transpose(out5, (2, 4, 0, 1, 3))
  return o7.reshape(b, f, dim)


# speedup vs baseline: 1.0011x; 1.0011x over previous
import functools

import jax
import jax.numpy as jnp
from jax import lax
from jax.experimental import pallas as pl
from jax.experimental.pallas import tpu as pltpu
from jax.experimental.pallas import tpu_sc as plsc

_LANES = 16


def _make_gather(batch: int, fields: int, dim: int, num_workers: int):
  per_w = batch // num_workers
  pitch = per_w + 1
  mesh = plsc.VectorSubcoreMesh(core_axis_name="c", subcore_axis_name="s")
  nc = mesh.num_cores

  @functools.partial(
      pl.kernel,
      out_type=jax.ShapeDtypeStruct(
          (fields, dim // 8, batch // 128, 8, 128), jnp.float32),
      mesh=mesh,
      scratch_types=[
          pltpu.VMEM((2, per_w), jnp.int32),
          pltpu.VMEM((2, per_w, dim), jnp.float32),
          pltpu.VMEM((2, dim, pitch), jnp.float32),
          pltpu.SemaphoreType.DMA,
          pltpu.SemaphoreType.DMA,
          pltpu.SemaphoreType.DMA,
          pltpu.SemaphoreType.DMA,
          pltpu.SemaphoreType.DMA,
          pltpu.SemaphoreType.DMA,
      ],
      compiler_params=pltpu.CompilerParams(
          use_tc_tiling_on_sc=False, needs_layout_passes=False,
          disable_bounds_checks=True),
  )
  def gather_kernel(table_hbm, idxt_hbm, out_hbm, idx_v, rows_v, cols_v,
                    is0, is1, gs0, gs1, ss0, ss1):
    wid = lax.axis_index("s") * nc + lax.axis_index("c")
    b0 = wid * per_w
    lane = lax.iota(jnp.int32, _LANES)
    isem = (is0, is1)
    gsem = (gs0, gs1)
    ssem = (ss0, ss1)

    def fire_idx(f, p):
      pltpu.async_copy(idxt_hbm.at[f, pl.ds(b0, per_w)], idx_v.at[p], isem[p])

    def wait_idx(p):
      pltpu.make_async_copy(idxt_hbm.at[0, pl.ds(b0, per_w)], idx_v.at[p],
                            isem[p]).wait()

    def fire_gather(p):
      pltpu.async_copy(table_hbm.at[idx_v.at[p]], rows_v.at[p], gsem[p])

    def wait_gather(p):
      pltpu.make_async_copy(table_hbm.at[idx_v.at[p]], rows_v.at[p],
                            gsem[p]).wait()

    def fire_store(f, p):
      for dt in range(dim // 8):
        for btr in range(per_w // 128):
          pltpu.async_copy(
              cols_v.at[p, pl.ds(dt * 8, 8), pl.ds(btr * 128, 128)],
              out_hbm.at[f, dt, wid * (per_w // 128) + btr, :, :], ssem[p])

    def wait_store(p):
      for dt in range(dim // 8):
        for btr in range(per_w // 128):
          pltpu.make_async_copy(
              cols_v.at[p, pl.ds(dt * 8, 8), pl.ds(btr * 128, 128)],
              out_hbm.at[0, dt, wid * (per_w // 128) + btr, :, :],
              ssem[p]).wait()

    def transpose(p):
      @plsc.parallel_loop(0, per_w, step=1, unroll=8)
      def _t_loop(r):
        rsplat = jnp.full((_LANES,), 0, jnp.int32) + r
        for h in range(dim // _LANES):
          d_ids = lane + h * _LANES
          v = rows_v[p, r, pl.ds(h * _LANES, _LANES)]
          plsc.store_scatter(cols_v.at[p], [d_ids, rsplat], v)

    fire_idx(0, 0)
    fire_idx(1, 1)
    wait_idx(0)
    fire_gather(0)

    @pl.loop(0, fields // 2)
    def _o_loop(o):
      for q in (0, 1):
        f = 2 * o + q
        p = q
        @pl.when(f < fields - 1)
        def _():
          wait_idx(1 - p)
          fire_gather(1 - p)

        wait_gather(p)

        @pl.when(f >= 2)
        def _():
          wait_store(p)

        transpose(p)
        fire_store(f, p)

        @pl.when(f < fields - 2)
        def _():
          fire_idx(f + 2, p)

    wait_store(0)
    wait_store(1)

  return gather_kernel


def kernel(x, emb_weight):
  b, f = x.shape
  dim = emb_weight.shape[1]
  xt = jnp.transpose(x)
  out5 = _make_gather(b, f, dim, 32)(emb_weight, xt)
  o7 = jnp.transpose(out5, (2, 4, 0, 1, 3))
  return o7.reshape(b, f, dim)
